# baseline (speedup 1.0000x reference)
import jax
import jax.numpy as jnp
from jax import lax
from jax.experimental import pallas as pl
from jax.experimental.pallas import tpu as pltpu

_VOCAB = 1000000
_EMBED = 32
_BATCH = 4096

_BM = 256
_NBLK = _BATCH // _BM


def _extract(stage, offs_col):
    onehot = jnp.where(
        offs_col[:, :, None] == lax.broadcasted_iota(jnp.int32, (1, 1, 128), 2),
        1.0, 0.0)
    return jnp.sum(stage * onehot, axis=2)


def _body(cw_smem, xw_smem, cwv_ref, xwv_ref, vt_ref, ut_ref, out_ref,
          xe_t, ce_rows, stage_x, stage_c, sem_x, sem_c):
    b = pl.program_id(0)

    def issue_round(tbl_ref, idx_smem, stage_buf, sem, base):
        def one(k, carry):
            idx = idx_smem[base + k]
            cb = pl.multiple_of((idx >> 7) << 7, 128)
            pltpu.make_async_copy(
                tbl_ref.at[:, pl.ds(cb, 128)], stage_buf.at[k], sem
            ).start()
            return carry
        lax.fori_loop(0, _BM, one, 0)

    def drain_round(tbl_ref, stage_buf, sem):
        def one(k, carry):
            pltpu.make_async_copy(
                tbl_ref.at[:, pl.ds(0, 128)], stage_buf.at[k], sem
            ).wait()
            return carry
        lax.fori_loop(0, _BM, one, 0)

    @pl.when(b == 0)
    def _build_xe():
        issue_round(ut_ref, xw_smem, stage_x.at[0], sem_x, 0)
        for r in range(_NBLK):
            if r + 1 < _NBLK:
                issue_round(ut_ref, xw_smem, stage_x.at[(r + 1) % 2], sem_x,
                            (r + 1) * _BM)
            drain_round(ut_ref, stage_x.at[r % 2], sem_x)
            offs = xwv_ref[pl.ds(r * _BM, _BM)] & 127
            e_rows = _extract(stage_x[r % 2], offs)
            xe_t[:, pl.ds(r * _BM, _BM)] = e_rows.T
        issue_round(vt_ref, cw_smem, stage_c.at[0], sem_c, 0)
        drain_round(vt_ref, stage_c.at[0], sem_c)

    offs_c = cwv_ref[...] & 127
    ce_rows[...] = _extract(stage_c[b % 2], offs_c)

    @pl.when(b + 1 < _NBLK)
    def _prefetch_next_ce():
        issue_round(vt_ref, cw_smem, stage_c.at[(b + 1) % 2], sem_c,
                    (b + 1) * _BM)

    scores = lax.dot_general(
        ce_rows[...], xe_t[...], (((1,), (0,)), ((), ())),
        preferred_element_type=jnp.float32)
    m = jnp.max(scores, axis=1, keepdims=True)
    s = jnp.sum(jnp.exp(scores - m), axis=1, keepdims=True)
    out_ref[...] = scores - (m + jnp.log(s))

    @pl.when(b + 1 < _NBLK)
    def _drain_next_ce():
        drain_round(vt_ref, stage_c.at[(b + 1) % 2], sem_c)


def kernel(center_words, context_words, embedding_v, embedding_u):
    cw = center_words.astype(jnp.int32)
    xw = context_words.astype(jnp.int32)
    vt = embedding_v.T
    ut = embedding_u.T
    cwv = cw.reshape(_BATCH, 1)
    xwv = xw.reshape(_BATCH, 1)
    return pl.pallas_call(
        _body,
        grid=(_NBLK,),
        in_specs=[
            pl.BlockSpec(memory_space=pltpu.SMEM),
            pl.BlockSpec(memory_space=pltpu.SMEM),
            pl.BlockSpec((_BM, 1), lambda i: (i, 0)),
            pl.BlockSpec((_BATCH, 1), lambda i: (0, 0)),
            pl.BlockSpec(memory_space=pl.ANY),
            pl.BlockSpec(memory_space=pl.ANY),
        ],
        out_specs=pl.BlockSpec((_BM, _BATCH), lambda i: (i, 0)),
        out_shape=jax.ShapeDtypeStruct((_BATCH, _BATCH), jnp.float32),
        scratch_shapes=[
            pltpu.VMEM((_EMBED, _BATCH), jnp.float32),
            pltpu.VMEM((_BM, _EMBED), jnp.float32),
            pltpu.VMEM((2, _BM, _EMBED, 128), jnp.float32),
            pltpu.VMEM((2, _BM, _EMBED, 128), jnp.float32),
            pltpu.SemaphoreType.DMA,
            pltpu.SemaphoreType.DMA,
        ],
    )(cw, xw, cwv, xwv, vt, ut)


# speedup vs baseline: 1.7098x; 1.7098x over previous
import jax
import jax.numpy as jnp
from jax import lax
from jax.experimental import pallas as pl
from jax.experimental.pallas import tpu as pltpu

_VOCAB = 1000000
_EMBED = 32
_BATCH = 4096

_BM = 256
_NBLK = _BATCH // _BM


def _extract(stage, offs_col):
    onehot = jnp.where(
        offs_col[:, :, None] == lax.broadcasted_iota(jnp.int32, (1, 1, 128), 2),
        1.0, 0.0)
    return jnp.sum(stage * onehot, axis=2)


def _body(cw_smem, xw_smem, cwv_ref, xwv_ref, vt_ref, ut_ref, out_ref,
          xe_t, ce_rows, stage_x, stage_c, sem_x, sem_c):
    b = pl.program_id(0)

    def issue_round(tbl_ref, idx_smem, stage_buf, sem, base):
        def one(k, carry):
            idx = idx_smem[base + k]
            cb = pl.multiple_of((idx >> 7) << 7, 128)
            pltpu.make_async_copy(
                tbl_ref.at[:, pl.ds(cb, 128)], stage_buf.at[k], sem
            ).start()
            return carry
        lax.fori_loop(0, _BM, one, 0, unroll=8)

    def drain_round(tbl_ref, stage_buf, sem):
        def one(k, carry):
            pltpu.make_async_copy(
                tbl_ref.at[:, pl.ds(0, 128)], stage_buf.at[k], sem
            ).wait()
            return carry
        lax.fori_loop(0, _BM, one, 0, unroll=8)

    @pl.when(b == 0)
    def _build_xe():
        issue_round(ut_ref, xw_smem, stage_x.at[0], sem_x, 0)
        for r in range(_NBLK):
            if r + 1 < _NBLK:
                issue_round(ut_ref, xw_smem, stage_x.at[(r + 1) % 2], sem_x,
                            (r + 1) * _BM)
            drain_round(ut_ref, stage_x.at[r % 2], sem_x)
            offs = xwv_ref[pl.ds(r * _BM, _BM)] & 127
            e_rows = _extract(stage_x[r % 2], offs)
            xe_t[:, pl.ds(r * _BM, _BM)] = e_rows.T
        issue_round(vt_ref, cw_smem, stage_c.at[0], sem_c, 0)
        drain_round(vt_ref, stage_c.at[0], sem_c)

    offs_c = cwv_ref[...] & 127
    ce_rows[...] = _extract(stage_c[b % 2], offs_c)

    @pl.when(b + 1 < _NBLK)
    def _prefetch_next_ce():
        issue_round(vt_ref, cw_smem, stage_c.at[(b + 1) % 2], sem_c,
                    (b + 1) * _BM)

    scores = lax.dot_general(
        ce_rows[...], xe_t[...], (((1,), (0,)), ((), ())),
        preferred_element_type=jnp.float32)
    m = jnp.max(scores, axis=1, keepdims=True)
    s = jnp.sum(jnp.exp(scores - m), axis=1, keepdims=True)
    out_ref[...] = scores - (m + jnp.log(s))

    @pl.when(b + 1 < _NBLK)
    def _drain_next_ce():
        drain_round(vt_ref, stage_c.at[(b + 1) % 2], sem_c)


def kernel(center_words, context_words, embedding_v, embedding_u):
    cw = center_words.astype(jnp.int32)
    xw = context_words.astype(jnp.int32)
    vt = embedding_v.T
    ut = embedding_u.T
    cwv = cw.reshape(_BATCH, 1)
    xwv = xw.reshape(_BATCH, 1)
    return pl.pallas_call(
        _body,
        grid=(_NBLK,),
        in_specs=[
            pl.BlockSpec(memory_space=pltpu.SMEM),
            pl.BlockSpec(memory_space=pltpu.SMEM),
            pl.BlockSpec((_BM, 1), lambda i: (i, 0)),
            pl.BlockSpec((_BATCH, 1), lambda i: (0, 0)),
            pl.BlockSpec(memory_space=pl.ANY),
            pl.BlockSpec(memory_space=pl.ANY),
        ],
        out_specs=pl.BlockSpec((_BM, _BATCH), lambda i: (i, 0)),
        out_shape=jax.ShapeDtypeStruct((_BATCH, _BATCH), jnp.float32),
        scratch_shapes=[
            pltpu.VMEM((_EMBED, _BATCH), jnp.float32),
            pltpu.VMEM((_BM, _EMBED), jnp.float32),
            pltpu.VMEM((2, _BM, _EMBED, 128), jnp.float32),
            pltpu.VMEM((2, _BM, _EMBED, 128), jnp.float32),
            pltpu.SemaphoreType.DMA,
            pltpu.SemaphoreType.DMA,
        ],
    )(cw, xw, cwv, xwv, vt, ut)


# speedup vs baseline: 1.7770x; 1.0393x over previous
import jax
import jax.numpy as jnp
from jax import lax
from jax.experimental import pallas as pl
from jax.experimental.pallas import tpu as pltpu

_VOCAB = 1000000
_EMBED = 32
_BATCH = 4096

_BM = 256
_NBLK = _BATCH // _BM


def _extract(stage, offs_col):
    onehot = jnp.where(
        offs_col[:, :, None] == lax.broadcasted_iota(jnp.int32, (1, 1, 128), 2),
        1.0, 0.0)
    return jnp.sum(stage * onehot, axis=2)


def _body(cw_smem, xw_smem, cwv_ref, xwv_ref, vt_ref, ut_ref, out_ref,
          xe_t, ce_rows, stage_x, stage_c, sem_x, sem_c):
    b = pl.program_id(0)

    def issue_round(tbl_ref, idx_smem, stage_buf, sem, base):
        def one(k, carry):
            idx = idx_smem[base + k]
            cb = pl.multiple_of((idx >> 7) << 7, 128)
            pltpu.make_async_copy(
                tbl_ref.at[:, pl.ds(cb, 128)], stage_buf.at[k], sem
            ).start()
            return carry
        lax.fori_loop(0, _BM, one, 0, unroll=16)

    def drain_round(tbl_ref, stage_buf, sem):
        def one(k, carry):
            pltpu.make_async_copy(
                tbl_ref.at[:, pl.ds(0, 128)], stage_buf.at[k], sem
            ).wait()
            return carry
        lax.fori_loop(0, _BM, one, 0, unroll=16)

    @pl.when(b == 0)
    def _build_xe():
        issue_round(ut_ref, xw_smem, stage_x.at[0], sem_x, 0)
        for r in range(_NBLK):
            if r + 1 < _NBLK:
                issue_round(ut_ref, xw_smem, stage_x.at[(r + 1) % 2], sem_x,
                            (r + 1) * _BM)
            drain_round(ut_ref, stage_x.at[r % 2], sem_x)
            offs = xwv_ref[pl.ds(r * _BM, _BM)] & 127
            e_rows = _extract(stage_x[r % 2], offs)
            xe_t[:, pl.ds(r * _BM, _BM)] = e_rows.T
        issue_round(vt_ref, cw_smem, stage_c.at[0], sem_c, 0)
        drain_round(vt_ref, stage_c.at[0], sem_c)

    offs_c = cwv_ref[...] & 127
    ce_rows[...] = _extract(stage_c[b % 2], offs_c)

    @pl.when(b + 1 < _NBLK)
    def _prefetch_next_ce():
        issue_round(vt_ref, cw_smem, stage_c.at[(b + 1) % 2], sem_c,
                    (b + 1) * _BM)

    scores = lax.dot_general(
        ce_rows[...], xe_t[...], (((1,), (0,)), ((), ())),
        preferred_element_type=jnp.float32)
    m = jnp.max(scores, axis=1, keepdims=True)
    s = jnp.sum(jnp.exp(scores - m), axis=1, keepdims=True)
    out_ref[...] = scores - (m + jnp.log(s))

    @pl.when(b + 1 < _NBLK)
    def _drain_next_ce():
        drain_round(vt_ref, stage_c.at[(b + 1) % 2], sem_c)


def kernel(center_words, context_words, embedding_v, embedding_u):
    cw = center_words.astype(jnp.int32)
    xw = context_words.astype(jnp.int32)
    vt = embedding_v.T
    ut = embedding_u.T
    cwv = cw.reshape(_BATCH, 1)
    xwv = xw.reshape(_BATCH, 1)
    return pl.pallas_call(
        _body,
        grid=(_NBLK,),
        in_specs=[
            pl.BlockSpec(memory_space=pltpu.SMEM),
            pl.BlockSpec(memory_space=pltpu.SMEM),
            pl.BlockSpec((_BM, 1), lambda i: (i, 0)),
            pl.BlockSpec((_BATCH, 1), lambda i: (0, 0)),
            pl.BlockSpec(memory_space=pl.ANY),
            pl.BlockSpec(memory_space=pl.ANY),
        ],
        out_specs=pl.BlockSpec((_BM, _BATCH), lambda i: (i, 0)),
        out_shape=jax.ShapeDtypeStruct((_BATCH, _BATCH), jnp.float32),
        scratch_shapes=[
            pltpu.VMEM((_EMBED, _BATCH), jnp.float32),
            pltpu.VMEM((_BM, _EMBED), jnp.float32),
            pltpu.VMEM((2, _BM, _EMBED, 128), jnp.float32),
            pltpu.VMEM((2, _BM, _EMBED, 128), jnp.float32),
            pltpu.SemaphoreType.DMA,
            pltpu.SemaphoreType.DMA,
        ],
    )(cw, xw, cwv, xwv, vt, ut)
